# baseline (device time: 24032 ns/iter reference)
import jax
import jax.numpy as jnp
from jax import lax
from jax.experimental import pallas as pl
from jax.experimental.pallas import tpu as pltpu

N_DEV = 32
GROUP = 8
N_GROUPS = N_DEV // GROUP
N_CHUNKS = 2


def kernel(x, gamma, beta):
    m, n_per = x.shape
    n_total = n_per * N_DEV
    mc = m // N_CHUNKS

    def body(
        x_ref, gamma_ref, beta_ref, out_ref,
        buf1_ref, buf2_ref, s1_send, s1_recv, s2_send, s2_recv,
    ):
        my = lax.axis_index("i")
        g = my // GROUP
        idx = my % GROUP

        def peer1(d):
            return g * GROUP + (idx + d) % GROUP

        def peer2(e):
            return ((g + e) % N_GROUPS) * GROUP + idx

        barrier_sem = pltpu.get_barrier_semaphore()
        for d in range(1, GROUP):
            pl.semaphore_signal(
                barrier_sem, inc=1,
                device_id=(peer1(d),), device_id_type=pl.DeviceIdType.MESH,
            )
        for e in range(1, N_GROUPS):
            pl.semaphore_signal(
                barrier_sem, inc=1,
                device_id=(peer2(e),), device_id_type=pl.DeviceIdType.MESH,
            )

        def stage1_rdma(c, d):
            return pltpu.make_async_remote_copy(
                src_ref=buf1_ref.at[c, 0],
                dst_ref=buf1_ref.at[c, d],
                send_sem=s1_send.at[c * GROUP + d],
                recv_sem=s1_recv.at[c * GROUP + d],
                device_id=(peer1(d),),
                device_id_type=pl.DeviceIdType.MESH,
            )

        def stage2_rdma(c, e):
            return pltpu.make_async_remote_copy(
                src_ref=buf2_ref.at[c, 0],
                dst_ref=buf2_ref.at[c, e],
                send_sem=s2_send.at[c * N_GROUPS + e],
                recv_sem=s2_recv.at[c * N_GROUPS + e],
                device_id=(peer2(e),),
                device_id_type=pl.DeviceIdType.MESH,
            )

        xs = []
        for c in range(N_CHUNKS):
            x_c = x_ref[pl.ds(c * mc, mc), :]
            xs.append(x_c)
            buf1_ref[c, 0, 0, :] = jnp.sum(x_c, axis=1)
            buf1_ref[c, 0, 1, :] = jnp.sum(x_c * x_c, axis=1)

        pl.semaphore_wait(barrier_sem, GROUP - 1 + N_GROUPS - 1)

        st1 = [[stage1_rdma(c, d) for d in range(1, GROUP)] for c in range(N_CHUNKS)]
        for c in range(N_CHUNKS):
            for rdma in st1[c]:
                rdma.start()

        st2 = []
        for c in range(N_CHUNKS):
            for rdma in st1[c]:
                rdma.wait()
            buf2_ref[c, 0, :, :] = jnp.sum(buf1_ref[c, :, :, :], axis=0)
            rdmas = [stage2_rdma(c, e) for e in range(1, N_GROUPS)]
            for rdma in rdmas:
                rdma.start()
            st2.append(rdmas)

        for c in range(N_CHUNKS):
            for rdma in st2[c]:
                rdma.wait()
            totals = jnp.sum(buf2_ref[c, :, :, :], axis=0)
            mean = totals[0, :] / n_total
            var = totals[1, :] / n_total - mean * mean
            inv = lax.rsqrt(var + 1e-5)
            normed = (xs[c] - mean[:, None]) * inv[:, None]
            out_ref[pl.ds(c * mc, mc), :] = (
                gamma_ref[:, :] * normed + beta_ref[:, :]
            )

    return pl.pallas_call(
        body,
        out_shape=jax.ShapeDtypeStruct((m, n_per), jnp.float32),
        in_specs=[
            pl.BlockSpec(memory_space=pltpu.VMEM),
            pl.BlockSpec(memory_space=pltpu.VMEM),
            pl.BlockSpec(memory_space=pltpu.VMEM),
        ],
        out_specs=pl.BlockSpec(memory_space=pltpu.VMEM),
        scratch_shapes=[
            pltpu.VMEM((N_CHUNKS, GROUP, 2, mc), jnp.float32),
            pltpu.VMEM((N_CHUNKS, N_GROUPS, 2, mc), jnp.float32),
            pltpu.SemaphoreType.DMA((N_CHUNKS * GROUP,)),
            pltpu.SemaphoreType.DMA((N_CHUNKS * GROUP,)),
            pltpu.SemaphoreType.DMA((N_CHUNKS * N_GROUPS,)),
            pltpu.SemaphoreType.DMA((N_CHUNKS * N_GROUPS,)),
        ],
        compiler_params=pltpu.CompilerParams(collective_id=0),
    )(x, gamma.reshape(1, n_per), beta.reshape(1, n_per))


# device time: 23992 ns/iter; 1.0017x vs baseline; 1.0017x over previous
import jax
import jax.numpy as jnp
from jax import lax
from jax.experimental import pallas as pl
from jax.experimental.pallas import tpu as pltpu

N_DEV = 32
GROUP = 8
N_GROUPS = N_DEV // GROUP


def kernel(x, gamma, beta):
    m, n_per = x.shape
    n_total = n_per * N_DEV

    def body(
        x_ref, gamma_ref, beta_ref, out_ref,
        buf1_ref, buf2_ref, s1_send, s1_recv, s2_send, s2_recv,
    ):
        my = lax.axis_index("i")
        g = my // GROUP
        idx = my % GROUP

        barrier_sem = pltpu.get_barrier_semaphore()
        for d in range(1, GROUP):
            pl.semaphore_signal(
                barrier_sem, inc=1,
                device_id=(g * GROUP + (idx + d) % GROUP,),
                device_id_type=pl.DeviceIdType.MESH,
            )
        for e in range(1, N_GROUPS):
            pl.semaphore_signal(
                barrier_sem, inc=1,
                device_id=(((g + e) % N_GROUPS) * GROUP + idx,),
                device_id_type=pl.DeviceIdType.MESH,
            )

        x_val = x_ref[:, :]
        buf1_ref[0, 0, :] = jnp.sum(x_val, axis=1)
        buf1_ref[0, 1, :] = jnp.sum(x_val * x_val, axis=1)

        pl.semaphore_wait(barrier_sem, GROUP - 1 + N_GROUPS - 1)

        st1 = []
        for d in range(1, GROUP):
            rdma = pltpu.make_async_remote_copy(
                src_ref=buf1_ref.at[0],
                dst_ref=buf1_ref.at[d],
                send_sem=s1_send.at[d],
                recv_sem=s1_recv.at[d],
                device_id=(g * GROUP + (idx + d) % GROUP,),
                device_id_type=pl.DeviceIdType.MESH,
            )
            rdma.start()
            st1.append(rdma)
        for rdma in st1:
            rdma.wait()

        buf2_ref[0, :, :] = jnp.sum(buf1_ref[:, :, :], axis=0)

        st2 = []
        for e in range(1, N_GROUPS):
            rdma = pltpu.make_async_remote_copy(
                src_ref=buf2_ref.at[0],
                dst_ref=buf2_ref.at[e],
                send_sem=s2_send.at[e],
                recv_sem=s2_recv.at[e],
                device_id=(((g + e) % N_GROUPS) * GROUP + idx,),
                device_id_type=pl.DeviceIdType.MESH,
            )
            rdma.start()
            st2.append(rdma)
        for rdma in st2:
            rdma.wait()

        totals = jnp.sum(buf2_ref[:, :, :], axis=0)
        mean = totals[0, :] / n_total
        var = totals[1, :] / n_total - mean * mean
        inv = lax.rsqrt(var + 1e-5)

        normed = (x_val - mean[:, None]) * inv[:, None]
        out_ref[:, :] = gamma_ref[:, :] * normed + beta_ref[:, :]

    return pl.pallas_call(
        body,
        out_shape=jax.ShapeDtypeStruct((m, n_per), jnp.float32),
        in_specs=[
            pl.BlockSpec(memory_space=pltpu.VMEM),
            pl.BlockSpec(memory_space=pltpu.VMEM),
            pl.BlockSpec(memory_space=pltpu.VMEM),
        ],
        out_specs=pl.BlockSpec(memory_space=pltpu.VMEM),
        scratch_shapes=[
            pltpu.VMEM((GROUP, 2, m), jnp.float32),
            pltpu.VMEM((N_GROUPS, 2, m), jnp.float32),
            pltpu.SemaphoreType.DMA((GROUP,)),
            pltpu.SemaphoreType.DMA((GROUP,)),
            pltpu.SemaphoreType.DMA((N_GROUPS,)),
            pltpu.SemaphoreType.DMA((N_GROUPS,)),
        ],
        compiler_params=pltpu.CompilerParams(collective_id=0),
    )(x, gamma.reshape(1, n_per), beta.reshape(1, n_per))
